# baseline (speedup 1.0000x reference)
import functools

import jax
import jax.numpy as jnp
from jax import lax
from jax.experimental import pallas as pl
from jax.experimental.pallas import tpu as pltpu
from jax.experimental.pallas import tpu_sc as plsc

N = 10000
F = 16
P = 4
OUT = 256
HID = 128
ODIM = 12
E = 640000

NP_ = 10240
NROW = NP_ // 16
FB = 4
NC = 2
NS = 16
CH = 1600
MAGIC = 0x5F3759DF


def _fast_rsqrt(d):
    y = plsc.bitcast(MAGIC - (plsc.bitcast(d, jnp.int32) >> 1), jnp.float32)
    for _ in range(3):
        y = y * (1.5 - 0.5 * d * y * y)
    return y


def _sc_body(rowh, colh, ew, xt, out_hbm, dis_hbm,
             deg_v, dis_v, tmp_v, x_blk, out_blk,
             ra, ca, wa, rb, cb, wb, sema, semb,
             shared_part, shared_sum):
    cid = lax.axis_index("c")
    tid = lax.axis_index("s")
    zero16 = jnp.zeros((16,), jnp.float32)
    nslice = NP_ // NS
    sbase = tid * nslice

    @plsc.parallel_loop(0, NP_ // 16, 1, unroll=8)
    def _(i):
        deg_v[pl.ds(i * 16, 16)] = zero16

    e1base = tid * (E // NS)

    def p1_chunk(k, _):
        pltpu.sync_copy(colh.at[pl.ds(e1base + k * CH, CH)], cb)
        pltpu.sync_copy(ew.at[pl.ds(e1base + k * CH, CH)], wb)

        @plsc.parallel_loop(0, CH // 16, 1, unroll=4)
        def _(g):
            c16 = cb[pl.ds(g * 16, 16)]
            w16 = wb[pl.ds(g * 16, 16)]
            plsc.addupdate_scatter(deg_v, [c16], w16)
        return 0
    lax.fori_loop(0, (E // NS) // CH, p1_chunk, 0)

    pltpu.sync_copy(deg_v, shared_part.at[tid])
    plsc.subcore_barrier()

    def zero_acc(i, _):
        deg_v[pl.ds(sbase + i * 16, 16)] = zero16
        return 0
    lax.fori_loop(0, nslice // 16, zero_acc, 0)
    for k in range(NS):
        pltpu.sync_copy(shared_part.at[k, pl.ds(sbase, nslice)], tmp_v)

        def acc_add(i, _):
            a = deg_v[pl.ds(sbase + i * 16, 16)]
            deg_v[pl.ds(sbase + i * 16, 16)] = a + tmp_v[pl.ds(i * 16, 16)]
            return 0
        lax.fori_loop(0, nslice // 16, acc_add, 0)
    pltpu.sync_copy(deg_v.at[pl.ds(sbase, nslice)],
                    shared_sum.at[pl.ds(sbase, nslice)])
    plsc.subcore_barrier()
    pltpu.sync_copy(shared_sum, deg_v)

    def mk_dis(i, _):
        dis_v[pl.ds(i * 16, 16)] = _fast_rsqrt(deg_v[pl.ds(i * 16, 16)] + 1.0)
        return 0
    lax.fori_loop(0, NP_ // 16, mk_dis, 0)

    @pl.when((tid == 0) & (cid == 0))
    def _():
        pltpu.sync_copy(dis_v, dis_hbm)

    pltpu.sync_copy(xt.at[tid], x_blk)

    @plsc.parallel_loop(0, (NP_ * FB) // 16, 1, unroll=8)
    def _(i):
        out_blk[pl.ds(i * 16, 16)] = zero16

    e2base = cid * (E // NC)
    nch2 = (E // NC) // CH

    def _start(bufs, sem, cidx):
        off = e2base + cidx * CH
        pltpu.async_copy(rowh.at[pl.ds(off, CH)], bufs[0], sem)
        pltpu.async_copy(colh.at[pl.ds(off, CH)], bufs[1], sem)
        pltpu.async_copy(ew.at[pl.ds(off, CH)], bufs[2], sem)

    def _drain(bufs, sem):
        pltpu.make_async_copy(rowh.at[pl.ds(e2base, CH)], bufs[0], sem).wait()
        pltpu.make_async_copy(colh.at[pl.ds(e2base, CH)], bufs[1], sem).wait()
        pltpu.make_async_copy(ew.at[pl.ds(e2base, CH)], bufs[2], sem).wait()

    def _process(bufs):
        @plsc.parallel_loop(0, CH // 16, 1, unroll=10)
        def _(g):
            r16 = bufs[0][pl.ds(g * 16, 16)]
            c16 = bufs[1][pl.ds(g * 16, 16)]
            w16 = bufs[2][pl.ds(g * 16, 16)]
            dr = plsc.load_gather(dis_v, [r16])
            dc = plsc.load_gather(dis_v, [c16])
            nrm = w16 * dr * dc
            rb4 = r16 * FB
            cb4 = c16 * FB
            for j in range(FB):
                xv = plsc.load_gather(x_blk, [rb4 + j])
                plsc.addupdate_scatter(out_blk, [cb4 + j], xv * nrm)

    bufs_a = (ra, ca, wa)
    bufs_b = (rb, cb, wb)
    _start(bufs_a, sema, 0)

    def p2_pair(k, _):
        c0 = 2 * k
        _start(bufs_b, semb, c0 + 1)
        _drain(bufs_a, sema)
        _process(bufs_a)

        @pl.when(c0 + 2 < nch2)
        def _():
            _start(bufs_a, sema, c0 + 2)
        _drain(bufs_b, semb)
        _process(bufs_b)
        return 0
    lax.fori_loop(0, nch2 // 2, p2_pair, 0)

    pltpu.sync_copy(out_blk, out_hbm.at[cid, tid])


def _sc_spmm(edge_index, edge_attr, xt):
    mesh = plsc.VectorSubcoreMesh(core_axis_name="c", subcore_axis_name="s",
                                  num_cores=NC, num_subcores=NS)
    fn = pl.kernel(
        _sc_body,
        out_type=[
            jax.ShapeDtypeStruct((NC, NS, NP_ * FB), jnp.float32),
            jax.ShapeDtypeStruct((NP_,), jnp.float32),
        ],
        mesh=mesh,
        scratch_types=[
            pltpu.VMEM((NP_,), jnp.float32),
            pltpu.VMEM((NP_,), jnp.float32),
            pltpu.VMEM((NP_ // NS,), jnp.float32),
            pltpu.VMEM((NP_ * FB,), jnp.float32),
            pltpu.VMEM((NP_ * FB,), jnp.float32),
            pltpu.VMEM((CH,), jnp.int32),
            pltpu.VMEM((CH,), jnp.int32),
            pltpu.VMEM((CH,), jnp.float32),
            pltpu.VMEM((CH,), jnp.int32),
            pltpu.VMEM((CH,), jnp.int32),
            pltpu.VMEM((CH,), jnp.float32),
            pltpu.SemaphoreType.DMA,
            pltpu.SemaphoreType.DMA,
            pltpu.MemorySpace.VMEM_SHARED((NS, NP_), jnp.float32),
            pltpu.MemorySpace.VMEM_SHARED((NP_,), jnp.float32),
        ],
        compiler_params=pltpu.CompilerParams(needs_layout_passes=False),
    )
    return fn(edge_index[0], edge_index[1], edge_attr, xt)


def _tc_body(ax_ref, x_ref, dis_ref, wzc, wzl, bzc, bzl, whc, whl, bhc, bhl,
             att_ref, w1, b1, w2, b2, out_ref, hid_ref):
    parts = ax_ref[...]
    dis = dis_ref[...]
    ax = parts[0] + parts[1] + x_ref[...] * (dis * dis)

    mz = jnp.dot(wzc[...], wzl[...], preferred_element_type=jnp.float32)
    cz = jnp.dot(bzc[...], wzl[...], preferred_element_type=jnp.float32) + bzl[...]
    mh = jnp.dot(whc[...], whl[...], preferred_element_type=jnp.float32)
    ch = jnp.dot(bhc[...], whl[...], preferred_element_type=jnp.float32) + bhl[...]

    a = att_ref[...]
    e = jnp.exp(a - jnp.max(a))
    pr = e / jnp.sum(e)

    hacc = jnp.zeros(hid_ref.shape, jnp.float32)
    for p in range(P):
        axp = ax[:, p * F:(p + 1) * F]
        az = jnp.dot(axp, mz, preferred_element_type=jnp.float32) + cz
        ah = jnp.dot(axp, mh, preferred_element_type=jnp.float32) + ch
        hacc = hacc + pr[0, p] * (1.0 - jax.nn.sigmoid(az)) * jnp.tanh(ah)
    hid_ref[...] = hacc
    h = jax.nn.relu(hacc)
    h = jax.nn.relu(jnp.dot(h, w1[...], preferred_element_type=jnp.float32)
                    + b1[...])
    out_ref[...] = jnp.dot(h, w2[...], preferred_element_type=jnp.float32) \
        + b2[...]


def _tc_dense(axparts, x64, dis, Wz_c, Wz_lt, bz_c, bz_l, Wh_c, Wh_lt,
              bh_c, bh_l, att, W1, b1, W2, b2):
    BN = 1024
    grid = (NP_ // BN,)
    full = lambda shape: pl.BlockSpec(shape, lambda i: (0,) * len(shape))
    return pl.pallas_call(
        _tc_body,
        grid=grid,
        in_specs=[
            pl.BlockSpec((NC, BN, F * P), lambda i: (0, i, 0)),
            pl.BlockSpec((BN, F * P), lambda i: (i, 0)),
            pl.BlockSpec((BN, 1), lambda i: (i, 0)),
            full((F, OUT)), full((OUT, OUT)), full((1, OUT)), full((1, OUT)),
            full((F, OUT)), full((OUT, OUT)), full((1, OUT)), full((1, OUT)),
            full((1, P)),
            full((OUT, HID)), full((1, HID)), full((HID, ODIM)),
            full((1, ODIM)),
        ],
        out_specs=[
            pl.BlockSpec((BN, ODIM), lambda i: (i, 0)),
            pl.BlockSpec((BN, OUT), lambda i: (i, 0)),
        ],
        out_shape=[
            jax.ShapeDtypeStruct((NP_, ODIM), jnp.float32),
            jax.ShapeDtypeStruct((NP_, OUT), jnp.float32),
        ],
    )(axparts, x64, dis, Wz_c, Wz_lt, bz_c, bz_l, Wh_c, Wh_lt, bh_c, bh_l,
      att, W1, b1, W2, b2)


def kernel(x, edge_index, edge_attr, Wz_c, bz_c, Wr_c, br_c, Wh_c, bh_c,
           Wz_l, bz_l, Wr_l, br_l, Wh_l, bh_l, att, W1, b1, W2, b2):
    x64 = jnp.transpose(x, (0, 2, 1)).reshape(N, P * F)
    x64 = jnp.pad(x64, ((0, NP_ - N), (0, 0)))
    xt = x64.reshape(NP_, NS, FB).transpose(1, 0, 2).reshape(NS, NP_ * FB)

    parts, dis = _sc_spmm(edge_index, edge_attr, xt)

    axparts = parts.reshape(NC, NS, NP_, FB).transpose(0, 2, 1, 3) \
        .reshape(NC, NP_, F * P)
    out, hid = _tc_dense(
        axparts, x64, dis.reshape(NP_, 1),
        Wz_c, Wz_l[:OUT], bz_c.reshape(1, OUT), bz_l.reshape(1, OUT),
        Wh_c, Wh_l[:OUT], bh_c.reshape(1, OUT), bh_l.reshape(1, OUT),

        att.reshape(1, P), W1, b1.reshape(1, HID), W2, b2.reshape(1, ODIM))
    return (out[:N], hid[:N])


# speedup vs baseline: 1.0664x; 1.0664x over previous
import functools

import jax
import jax.numpy as jnp
from jax import lax
from jax.experimental import pallas as pl
from jax.experimental.pallas import tpu as pltpu
from jax.experimental.pallas import tpu_sc as plsc

N = 10000
F = 16
P = 4
OUT = 256
HID = 128
ODIM = 12
E = 640000

NP_ = 10240
NROW = NP_ // 16
FB = 4
NC = 2
NS = 16
CH = 1600
MAGIC = 0x5F3759DF


def _fast_rsqrt(d):
    y = plsc.bitcast(MAGIC - (plsc.bitcast(d, jnp.int32) >> 1), jnp.float32)
    for _ in range(3):
        y = y * (1.5 - 0.5 * d * y * y)
    return y


def _sc_body(rowh, colh, ew, xt, out_hbm, dis_hbm,
             deg_v, dis_v, tmp_v, x_blk, out_blk,
             ra, ca, wa, rb, cb, wb, sema, semb,
             shared_part, shared_sum):
    cid = lax.axis_index("c")
    tid = lax.axis_index("s")
    zero16 = jnp.zeros((16,), jnp.float32)
    nslice = NP_ // NS
    sbase = tid * nslice

    @plsc.parallel_loop(0, NP_ // 16, 1, unroll=8)
    def _(i):
        deg_v[pl.ds(i * 16, 16)] = zero16

    e1base = tid * (E // NS)

    def p1_chunk(k, _):
        pltpu.sync_copy(colh.at[pl.ds(e1base + k * CH, CH)], cb)
        pltpu.sync_copy(ew.at[pl.ds(e1base + k * CH, CH)], wb)

        @plsc.parallel_loop(0, CH // 16, 1, unroll=4)
        def _(g):
            c16 = cb[pl.ds(g * 16, 16)]
            w16 = wb[pl.ds(g * 16, 16)]
            plsc.addupdate_scatter(deg_v, [c16], w16)
        return 0
    lax.fori_loop(0, (E // NS) // CH, p1_chunk, 0)

    pltpu.sync_copy(deg_v, shared_part.at[tid])
    plsc.subcore_barrier()

    def zero_acc(i, _):
        deg_v[pl.ds(sbase + i * 16, 16)] = zero16
        return 0
    lax.fori_loop(0, nslice // 16, zero_acc, 0)
    for k in range(NS):
        pltpu.sync_copy(shared_part.at[k, pl.ds(sbase, nslice)], tmp_v)

        def acc_add(i, _):
            a = deg_v[pl.ds(sbase + i * 16, 16)]
            deg_v[pl.ds(sbase + i * 16, 16)] = a + tmp_v[pl.ds(i * 16, 16)]
            return 0
        lax.fori_loop(0, nslice // 16, acc_add, 0)
    pltpu.sync_copy(deg_v.at[pl.ds(sbase, nslice)],
                    shared_sum.at[pl.ds(sbase, nslice)])
    plsc.subcore_barrier()
    pltpu.sync_copy(shared_sum, deg_v)

    def mk_dis(i, _):
        dis_v[pl.ds(i * 16, 16)] = _fast_rsqrt(deg_v[pl.ds(i * 16, 16)] + 1.0)
        return 0
    lax.fori_loop(0, NP_ // 16, mk_dis, 0)

    @pl.when((tid == 0) & (cid == 0))
    def _():
        pltpu.sync_copy(dis_v, dis_hbm)

    pltpu.sync_copy(xt.at[tid], x_blk)

    @plsc.parallel_loop(0, (NP_ * FB) // 16, 1, unroll=8)
    def _(i):
        out_blk[pl.ds(i * 16, 16)] = zero16

    e2base = cid * (E // NC)
    nch2 = (E // NC) // CH

    def _start(bufs, sem, cidx):
        off = e2base + cidx * CH
        pltpu.async_copy(rowh.at[pl.ds(off, CH)], bufs[0], sem)
        pltpu.async_copy(colh.at[pl.ds(off, CH)], bufs[1], sem)
        pltpu.async_copy(ew.at[pl.ds(off, CH)], bufs[2], sem)

    def _drain(bufs, sem):
        pltpu.make_async_copy(rowh.at[pl.ds(e2base, CH)], bufs[0], sem).wait()
        pltpu.make_async_copy(colh.at[pl.ds(e2base, CH)], bufs[1], sem).wait()
        pltpu.make_async_copy(ew.at[pl.ds(e2base, CH)], bufs[2], sem).wait()

    def _process(bufs):
        @plsc.parallel_loop(0, CH // 16, 1, unroll=5)
        def _(g):
            r16 = bufs[0][pl.ds(g * 16, 16)]
            c16 = bufs[1][pl.ds(g * 16, 16)]
            w16 = bufs[2][pl.ds(g * 16, 16)]
            dr = plsc.load_gather(dis_v, [r16])
            dc = plsc.load_gather(dis_v, [c16])
            nrm = w16 * dr * dc
            rb4 = r16 * FB
            cb4 = c16 * FB
            for j in range(FB):
                xv = plsc.load_gather(x_blk, [rb4 + j])
                plsc.addupdate_scatter(out_blk, [cb4 + j], xv * nrm)

    bufs_a = (ra, ca, wa)
    bufs_b = (rb, cb, wb)
    _start(bufs_a, sema, 0)

    def p2_pair(k, _):
        c0 = 2 * k
        _start(bufs_b, semb, c0 + 1)
        _drain(bufs_a, sema)
        _process(bufs_a)

        @pl.when(c0 + 2 < nch2)
        def _():
            _start(bufs_a, sema, c0 + 2)
        _drain(bufs_b, semb)
        _process(bufs_b)
        return 0
    lax.fori_loop(0, nch2 // 2, p2_pair, 0)

    pltpu.sync_copy(out_blk, out_hbm.at[cid, tid])


def _sc_spmm(edge_index, edge_attr, xt):
    mesh = plsc.VectorSubcoreMesh(core_axis_name="c", subcore_axis_name="s",
                                  num_cores=NC, num_subcores=NS)
    fn = pl.kernel(
        _sc_body,
        out_type=[
            jax.ShapeDtypeStruct((NC, NS, NP_ * FB), jnp.float32),
            jax.ShapeDtypeStruct((NP_,), jnp.float32),
        ],
        mesh=mesh,
        scratch_types=[
            pltpu.VMEM((NP_,), jnp.float32),
            pltpu.VMEM((NP_,), jnp.float32),
            pltpu.VMEM((NP_ // NS,), jnp.float32),
            pltpu.VMEM((NP_ * FB,), jnp.float32),
            pltpu.VMEM((NP_ * FB,), jnp.float32),
            pltpu.VMEM((CH,), jnp.int32),
            pltpu.VMEM((CH,), jnp.int32),
            pltpu.VMEM((CH,), jnp.float32),
            pltpu.VMEM((CH,), jnp.int32),
            pltpu.VMEM((CH,), jnp.int32),
            pltpu.VMEM((CH,), jnp.float32),
            pltpu.SemaphoreType.DMA,
            pltpu.SemaphoreType.DMA,
            pltpu.MemorySpace.VMEM_SHARED((NS, NP_), jnp.float32),
            pltpu.MemorySpace.VMEM_SHARED((NP_,), jnp.float32),
        ],
        compiler_params=pltpu.CompilerParams(needs_layout_passes=False),
    )
    return fn(edge_index[0], edge_index[1], edge_attr, xt)


def _tc_body(ax_ref, x_ref, dis_ref, wzc, wzl, bzc, bzl, whc, whl, bhc, bhl,
             att_ref, w1, b1, w2, b2, out_ref, hid_ref):
    parts = ax_ref[...]
    dis = dis_ref[...]
    ax = parts[0] + parts[1] + x_ref[...] * (dis * dis)

    mz = jnp.dot(wzc[...], wzl[...], preferred_element_type=jnp.float32)
    cz = jnp.dot(bzc[...], wzl[...], preferred_element_type=jnp.float32) + bzl[...]
    mh = jnp.dot(whc[...], whl[...], preferred_element_type=jnp.float32)
    ch = jnp.dot(bhc[...], whl[...], preferred_element_type=jnp.float32) + bhl[...]

    a = att_ref[...]
    e = jnp.exp(a - jnp.max(a))
    pr = e / jnp.sum(e)

    hacc = jnp.zeros(hid_ref.shape, jnp.float32)
    for p in range(P):
        axp = ax[:, p * F:(p + 1) * F]
        az = jnp.dot(axp, mz, preferred_element_type=jnp.float32) + cz
        ah = jnp.dot(axp, mh, preferred_element_type=jnp.float32) + ch
        hacc = hacc + pr[0, p] * (1.0 - jax.nn.sigmoid(az)) * jnp.tanh(ah)
    hid_ref[...] = hacc
    h = jax.nn.relu(hacc)
    h = jax.nn.relu(jnp.dot(h, w1[...], preferred_element_type=jnp.float32)
                    + b1[...])
    out_ref[...] = jnp.dot(h, w2[...], preferred_element_type=jnp.float32) \
        + b2[...]


def _tc_dense(axparts, x64, dis, Wz_c, Wz_lt, bz_c, bz_l, Wh_c, Wh_lt,
              bh_c, bh_l, att, W1, b1, W2, b2):
    BN = 1024
    grid = (NP_ // BN,)
    full = lambda shape: pl.BlockSpec(shape, lambda i: (0,) * len(shape))
    return pl.pallas_call(
        _tc_body,
        grid=grid,
        in_specs=[
            pl.BlockSpec((NC, BN, F * P), lambda i: (0, i, 0)),
            pl.BlockSpec((BN, F * P), lambda i: (i, 0)),
            pl.BlockSpec((BN, 1), lambda i: (i, 0)),
            full((F, OUT)), full((OUT, OUT)), full((1, OUT)), full((1, OUT)),
            full((F, OUT)), full((OUT, OUT)), full((1, OUT)), full((1, OUT)),
            full((1, P)),
            full((OUT, HID)), full((1, HID)), full((HID, ODIM)),
            full((1, ODIM)),
        ],
        out_specs=[
            pl.BlockSpec((BN, ODIM), lambda i: (i, 0)),
            pl.BlockSpec((BN, OUT), lambda i: (i, 0)),
        ],
        out_shape=[
            jax.ShapeDtypeStruct((NP_, ODIM), jnp.float32),
            jax.ShapeDtypeStruct((NP_, OUT), jnp.float32),
        ],
    )(axparts, x64, dis, Wz_c, Wz_lt, bz_c, bz_l, Wh_c, Wh_lt, bh_c, bh_l,
      att, W1, b1, W2, b2)


def kernel(x, edge_index, edge_attr, Wz_c, bz_c, Wr_c, br_c, Wh_c, bh_c,
           Wz_l, bz_l, Wr_l, br_l, Wh_l, bh_l, att, W1, b1, W2, b2):
    x64 = jnp.transpose(x, (0, 2, 1)).reshape(N, P * F)
    x64 = jnp.pad(x64, ((0, NP_ - N), (0, 0)))
    xt = x64.reshape(NP_, NS, FB).transpose(1, 0, 2).reshape(NS, NP_ * FB)

    parts, dis = _sc_spmm(edge_index, edge_attr, xt)

    axparts = parts.reshape(NC, NS, NP_, FB).transpose(0, 2, 1, 3) \
        .reshape(NC, NP_, F * P)
    out, hid = _tc_dense(
        axparts, x64, dis.reshape(NP_, 1),
        Wz_c, Wz_l[:OUT], bz_c.reshape(1, OUT), bz_l.reshape(1, OUT),
        Wh_c, Wh_l[:OUT], bh_c.reshape(1, OUT), bh_l.reshape(1, OUT),

        att.reshape(1, P), W1, b1.reshape(1, HID), W2, b2.reshape(1, ODIM))
    return (out[:N], hid[:N])


# speedup vs baseline: 1.5004x; 1.4070x over previous
import functools

import jax
import jax.numpy as jnp
from jax import lax
from jax.experimental import pallas as pl
from jax.experimental.pallas import tpu as pltpu
from jax.experimental.pallas import tpu_sc as plsc

N = 10000
F = 16
P = 4
OUT = 256
HID = 128
ODIM = 12
E = 640000

NP_ = 10240
NROW = NP_ // 16
FB = 4
NC = 2
NS = 16
CH = 1600
MAGIC = 0x5F3759DF


def _fast_rsqrt(d):
    y = plsc.bitcast(MAGIC - (plsc.bitcast(d, jnp.int32) >> 1), jnp.float32)
    for _ in range(3):
        y = y * (1.5 - 0.5 * d * y * y)
    return y


def _sc_body(rowh, colh, ew, xt, out_hbm, dis_hbm,
             deg_v, dis_v, tmp_v, x_blk, out_blk,
             ra, ca, wa, rb, cb, wb, sema, semb,
             shared_part, shared_sum):
    cid = lax.axis_index("c")
    tid = lax.axis_index("s")
    zero16 = jnp.zeros((16,), jnp.float32)
    nslice = NP_ // NS
    sbase = tid * nslice

    @plsc.parallel_loop(0, NP_ // 16, 1, unroll=8)
    def _(i):
        deg_v[pl.ds(i * 16, 16)] = zero16

    e1base = tid * (E // NS)

    def p1_chunk(k, _):
        pltpu.sync_copy(colh.at[pl.ds(e1base + k * CH, CH)], cb)
        pltpu.sync_copy(ew.at[pl.ds(e1base + k * CH, CH)], wb)

        @plsc.parallel_loop(0, CH // 16, 1, unroll=4)
        def _(g):
            c16 = cb[pl.ds(g * 16, 16)]
            w16 = wb[pl.ds(g * 16, 16)]
            plsc.addupdate_scatter(deg_v, [c16], w16)
        return 0
    lax.fori_loop(0, (E // NS) // CH, p1_chunk, 0)

    pltpu.sync_copy(deg_v, shared_part.at[tid])
    plsc.subcore_barrier()

    def zero_acc(i, _):
        deg_v[pl.ds(sbase + i * 16, 16)] = zero16
        return 0
    lax.fori_loop(0, nslice // 16, zero_acc, 0)
    for k in range(NS):
        pltpu.sync_copy(shared_part.at[k, pl.ds(sbase, nslice)], tmp_v)

        def acc_add(i, _):
            a = deg_v[pl.ds(sbase + i * 16, 16)]
            deg_v[pl.ds(sbase + i * 16, 16)] = a + tmp_v[pl.ds(i * 16, 16)]
            return 0
        lax.fori_loop(0, nslice // 16, acc_add, 0)
    pltpu.sync_copy(deg_v.at[pl.ds(sbase, nslice)],
                    shared_sum.at[pl.ds(sbase, nslice)])
    plsc.subcore_barrier()
    pltpu.sync_copy(shared_sum, deg_v)

    def mk_dis(i, _):
        dis_v[pl.ds(i * 16, 16)] = _fast_rsqrt(deg_v[pl.ds(i * 16, 16)] + 1.0)
        return 0
    lax.fori_loop(0, NP_ // 16, mk_dis, 0)

    @pl.when((tid == 0) & (cid == 0))
    def _():
        pltpu.sync_copy(dis_v, dis_hbm)

    pltpu.sync_copy(xt.at[tid], x_blk)

    @plsc.parallel_loop(0, (NP_ * FB) // 16, 1, unroll=8)
    def _(i):
        out_blk[pl.ds(i * 16, 16)] = zero16

    e2base = cid * (E // NC)
    nch2 = (E // NC) // CH

    def _start(bufs, sem, cidx):
        off = e2base + cidx * CH
        pltpu.async_copy(rowh.at[pl.ds(off, CH)], bufs[0], sem)
        pltpu.async_copy(colh.at[pl.ds(off, CH)], bufs[1], sem)
        pltpu.async_copy(ew.at[pl.ds(off, CH)], bufs[2], sem)

    def _drain(bufs, sem):
        pltpu.make_async_copy(rowh.at[pl.ds(e2base, CH)], bufs[0], sem).wait()
        pltpu.make_async_copy(colh.at[pl.ds(e2base, CH)], bufs[1], sem).wait()
        pltpu.make_async_copy(ew.at[pl.ds(e2base, CH)], bufs[2], sem).wait()

    def _process(bufs):
        @plsc.parallel_loop(0, CH // 16, 1, unroll=5)
        def _(g):
            r16 = bufs[0][pl.ds(g * 16, 16)]
            c16 = bufs[1][pl.ds(g * 16, 16)]
            w16 = bufs[2][pl.ds(g * 16, 16)]
            dr = plsc.load_gather(dis_v, [r16])
            dc = plsc.load_gather(dis_v, [c16])
            nrm = w16 * dr * dc
            for j in range(FB):
                xv = plsc.load_gather(x_blk, [r16 + j * NP_])
                plsc.addupdate_scatter(out_blk, [c16 + j * NP_], xv * nrm)

    bufs_a = (ra, ca, wa)
    bufs_b = (rb, cb, wb)
    _start(bufs_a, sema, 0)

    def p2_pair(k, _):
        c0 = 2 * k
        _start(bufs_b, semb, c0 + 1)
        _drain(bufs_a, sema)
        _process(bufs_a)

        @pl.when(c0 + 2 < nch2)
        def _():
            _start(bufs_a, sema, c0 + 2)
        _drain(bufs_b, semb)
        _process(bufs_b)
        return 0
    lax.fori_loop(0, nch2 // 2, p2_pair, 0)

    pltpu.sync_copy(out_blk, out_hbm.at[cid, tid])


def _sc_spmm(edge_index, edge_attr, xt):
    mesh = plsc.VectorSubcoreMesh(core_axis_name="c", subcore_axis_name="s",
                                  num_cores=NC, num_subcores=NS)
    fn = pl.kernel(
        _sc_body,
        out_type=[
            jax.ShapeDtypeStruct((NC, NS, NP_ * FB), jnp.float32),
            jax.ShapeDtypeStruct((NP_,), jnp.float32),
        ],
        mesh=mesh,
        scratch_types=[
            pltpu.VMEM((NP_,), jnp.float32),
            pltpu.VMEM((NP_,), jnp.float32),
            pltpu.VMEM((NP_ // NS,), jnp.float32),
            pltpu.VMEM((NP_ * FB,), jnp.float32),
            pltpu.VMEM((NP_ * FB,), jnp.float32),
            pltpu.VMEM((CH,), jnp.int32),
            pltpu.VMEM((CH,), jnp.int32),
            pltpu.VMEM((CH,), jnp.float32),
            pltpu.VMEM((CH,), jnp.int32),
            pltpu.VMEM((CH,), jnp.int32),
            pltpu.VMEM((CH,), jnp.float32),
            pltpu.SemaphoreType.DMA,
            pltpu.SemaphoreType.DMA,
            pltpu.MemorySpace.VMEM_SHARED((NS, NP_), jnp.float32),
            pltpu.MemorySpace.VMEM_SHARED((NP_,), jnp.float32),
        ],
        compiler_params=pltpu.CompilerParams(needs_layout_passes=False),
    )
    return fn(edge_index[0], edge_index[1], edge_attr, xt)


def _tc_body(ax_ref, x_ref, dis_ref, wzc, wzl, bzc, bzl, whc, whl, bhc, bhl,
             att_ref, w1, b1, w2, b2, out_ref, hid_ref):
    parts = ax_ref[...]
    dis = dis_ref[...]
    ax = parts[0] + parts[1] + x_ref[...] * (dis * dis)

    mz = jnp.dot(wzc[...], wzl[...], preferred_element_type=jnp.float32)
    cz = jnp.dot(bzc[...], wzl[...], preferred_element_type=jnp.float32) + bzl[...]
    mh = jnp.dot(whc[...], whl[...], preferred_element_type=jnp.float32)
    ch = jnp.dot(bhc[...], whl[...], preferred_element_type=jnp.float32) + bhl[...]

    a = att_ref[...]
    e = jnp.exp(a - jnp.max(a))
    pr = e / jnp.sum(e)

    hacc = jnp.zeros(hid_ref.shape, jnp.float32)
    for p in range(P):
        axp = ax[:, p * F:(p + 1) * F]
        az = jnp.dot(axp, mz, preferred_element_type=jnp.float32) + cz
        ah = jnp.dot(axp, mh, preferred_element_type=jnp.float32) + ch
        hacc = hacc + pr[0, p] * (1.0 - jax.nn.sigmoid(az)) * jnp.tanh(ah)
    hid_ref[...] = hacc
    h = jax.nn.relu(hacc)
    h = jax.nn.relu(jnp.dot(h, w1[...], preferred_element_type=jnp.float32)
                    + b1[...])
    out_ref[...] = jnp.dot(h, w2[...], preferred_element_type=jnp.float32) \
        + b2[...]


def _tc_dense(axparts, x64, dis, Wz_c, Wz_lt, bz_c, bz_l, Wh_c, Wh_lt,
              bh_c, bh_l, att, W1, b1, W2, b2):
    BN = 1024
    grid = (NP_ // BN,)
    full = lambda shape: pl.BlockSpec(shape, lambda i: (0,) * len(shape))
    return pl.pallas_call(
        _tc_body,
        grid=grid,
        in_specs=[
            pl.BlockSpec((NC, BN, F * P), lambda i: (0, i, 0)),
            pl.BlockSpec((BN, F * P), lambda i: (i, 0)),
            pl.BlockSpec((BN, 1), lambda i: (i, 0)),
            full((F, OUT)), full((OUT, OUT)), full((1, OUT)), full((1, OUT)),
            full((F, OUT)), full((OUT, OUT)), full((1, OUT)), full((1, OUT)),
            full((1, P)),
            full((OUT, HID)), full((1, HID)), full((HID, ODIM)),
            full((1, ODIM)),
        ],
        out_specs=[
            pl.BlockSpec((BN, ODIM), lambda i: (i, 0)),
            pl.BlockSpec((BN, OUT), lambda i: (i, 0)),
        ],
        out_shape=[
            jax.ShapeDtypeStruct((NP_, ODIM), jnp.float32),
            jax.ShapeDtypeStruct((NP_, OUT), jnp.float32),
        ],
    )(axparts, x64, dis, Wz_c, Wz_lt, bz_c, bz_l, Wh_c, Wh_lt, bh_c, bh_l,
      att, W1, b1, W2, b2)


def kernel(x, edge_index, edge_attr, Wz_c, bz_c, Wr_c, br_c, Wh_c, bh_c,
           Wz_l, bz_l, Wr_l, br_l, Wh_l, bh_l, att, W1, b1, W2, b2):
    x64 = jnp.transpose(x, (0, 2, 1)).reshape(N, P * F)
    x64 = jnp.pad(x64, ((0, NP_ - N), (0, 0)))
    xt = x64.reshape(NP_, NS, FB).transpose(1, 2, 0).reshape(NS, NP_ * FB)

    parts, dis = _sc_spmm(edge_index, edge_attr, xt)

    axparts = parts.reshape(NC, NS, FB, NP_).transpose(0, 3, 1, 2) \
        .reshape(NC, NP_, F * P)
    out, hid = _tc_dense(
        axparts, x64, dis.reshape(NP_, 1),
        Wz_c, Wz_l[:OUT], bz_c.reshape(1, OUT), bz_l.reshape(1, OUT),
        Wh_c, Wh_l[:OUT], bh_c.reshape(1, OUT), bh_l.reshape(1, OUT),

        att.reshape(1, P), W1, b1.reshape(1, HID), W2, b2.reshape(1, ODIM))
    return (out[:N], hid[:N])


# speedup vs baseline: 1.5177x; 1.0115x over previous
import functools

import jax
import jax.numpy as jnp
from jax import lax
from jax.experimental import pallas as pl
from jax.experimental.pallas import tpu as pltpu
from jax.experimental.pallas import tpu_sc as plsc

N = 10000
F = 16
P = 4
OUT = 256
HID = 128
ODIM = 12
E = 640000

NP_ = 10240
NROW = NP_ // 16
FB = 4
NC = 2
NS = 16
CH = 2000
MAGIC = 0x5F3759DF


def _fast_rsqrt(d):
    y = plsc.bitcast(MAGIC - (plsc.bitcast(d, jnp.int32) >> 1), jnp.float32)
    for _ in range(3):
        y = y * (1.5 - 0.5 * d * y * y)
    return y


def _sc_body(rowh, colh, ew, xt, out_hbm, dis_hbm, norm_hbm,
             deg_v, dis_v, tmp_v, x_blk, out_blk,
             ra, ca, wa, rb, cb, wb, sema, semb,
             shared_part, shared_sum):
    cid = lax.axis_index("c")
    tid = lax.axis_index("s")
    zero16 = jnp.zeros((16,), jnp.float32)
    nslice = NP_ // NS
    sbase = tid * nslice

    @plsc.parallel_loop(0, NP_ // 16, 1, unroll=8)
    def _(i):
        deg_v[pl.ds(i * 16, 16)] = zero16

    e1base = tid * (E // NS)

    def p1_chunk(k, _):
        pltpu.sync_copy(colh.at[pl.ds(e1base + k * CH, CH)], cb)
        pltpu.sync_copy(ew.at[pl.ds(e1base + k * CH, CH)], wb)

        @plsc.parallel_loop(0, CH // 16, 1, unroll=4)
        def _(g):
            c16 = cb[pl.ds(g * 16, 16)]
            w16 = wb[pl.ds(g * 16, 16)]
            plsc.addupdate_scatter(deg_v, [c16], w16)
        return 0
    lax.fori_loop(0, (E // NS) // CH, p1_chunk, 0)

    pltpu.sync_copy(deg_v, shared_part.at[tid])
    plsc.subcore_barrier()

    def zero_acc(i, _):
        deg_v[pl.ds(sbase + i * 16, 16)] = zero16
        return 0
    lax.fori_loop(0, nslice // 16, zero_acc, 0)
    for k in range(NS):
        pltpu.sync_copy(shared_part.at[k, pl.ds(sbase, nslice)], tmp_v)

        def acc_add(i, _):
            a = deg_v[pl.ds(sbase + i * 16, 16)]
            deg_v[pl.ds(sbase + i * 16, 16)] = a + tmp_v[pl.ds(i * 16, 16)]
            return 0
        lax.fori_loop(0, nslice // 16, acc_add, 0)
    pltpu.sync_copy(deg_v.at[pl.ds(sbase, nslice)],
                    shared_sum.at[pl.ds(sbase, nslice)])
    plsc.subcore_barrier()
    pltpu.sync_copy(shared_sum, deg_v)

    def mk_dis(i, _):
        dis_v[pl.ds(i * 16, 16)] = _fast_rsqrt(deg_v[pl.ds(i * 16, 16)] + 1.0)
        return 0
    lax.fori_loop(0, NP_ // 16, mk_dis, 0)

    @pl.when((tid == 0) & (cid == 0))
    def _():
        pltpu.sync_copy(dis_v, dis_hbm)

    nbase = cid * (E // NC) + tid * (E // (NC * NS))

    def p15_chunk(k, _):
        off = nbase + k * CH
        pltpu.sync_copy(rowh.at[pl.ds(off, CH)], ra)
        pltpu.sync_copy(colh.at[pl.ds(off, CH)], ca)
        pltpu.sync_copy(ew.at[pl.ds(off, CH)], wa)

        @plsc.parallel_loop(0, CH // 16, 1, unroll=5)
        def _(g):
            r16 = ra[pl.ds(g * 16, 16)]
            c16 = ca[pl.ds(g * 16, 16)]
            w16 = wa[pl.ds(g * 16, 16)]
            dr = plsc.load_gather(dis_v, [r16])
            dc = plsc.load_gather(dis_v, [c16])
            wb[pl.ds(g * 16, 16)] = w16 * dr * dc
        pltpu.sync_copy(wb, norm_hbm.at[pl.ds(off, CH)])
        return 0
    lax.fori_loop(0, (E // (NC * NS)) // CH, p15_chunk, 0)
    plsc.subcore_barrier()

    pltpu.sync_copy(xt.at[tid], x_blk)

    @plsc.parallel_loop(0, (NP_ * FB) // 16, 1, unroll=8)
    def _(i):
        out_blk[pl.ds(i * 16, 16)] = zero16

    e2base = cid * (E // NC)
    nch2 = (E // NC) // CH

    def _start(bufs, sem, cidx):
        off = e2base + cidx * CH
        pltpu.async_copy(rowh.at[pl.ds(off, CH)], bufs[0], sem)
        pltpu.async_copy(colh.at[pl.ds(off, CH)], bufs[1], sem)
        pltpu.async_copy(norm_hbm.at[pl.ds(off, CH)], bufs[2], sem)

    def _drain(bufs, sem):
        pltpu.make_async_copy(rowh.at[pl.ds(e2base, CH)], bufs[0], sem).wait()
        pltpu.make_async_copy(colh.at[pl.ds(e2base, CH)], bufs[1], sem).wait()
        pltpu.make_async_copy(ew.at[pl.ds(e2base, CH)], bufs[2], sem).wait()

    def _process(bufs):
        @plsc.parallel_loop(0, CH // 16, 1, unroll=5)
        def _(g):
            r16 = bufs[0][pl.ds(g * 16, 16)]
            c16 = bufs[1][pl.ds(g * 16, 16)]
            nrm = bufs[2][pl.ds(g * 16, 16)]
            for j in range(FB):
                xv = plsc.load_gather(x_blk, [r16 + j * NP_])
                plsc.addupdate_scatter(out_blk, [c16 + j * NP_], xv * nrm)

    bufs_a = (ra, ca, wa)
    bufs_b = (rb, cb, wb)
    _start(bufs_a, sema, 0)

    def p2_pair(k, _):
        c0 = 2 * k
        _start(bufs_b, semb, c0 + 1)
        _drain(bufs_a, sema)
        _process(bufs_a)

        @pl.when(c0 + 2 < nch2)
        def _():
            _start(bufs_a, sema, c0 + 2)
        _drain(bufs_b, semb)
        _process(bufs_b)
        return 0
    lax.fori_loop(0, nch2 // 2, p2_pair, 0)

    pltpu.sync_copy(out_blk, out_hbm.at[cid, tid])


def _sc_spmm(edge_index, edge_attr, xt):
    mesh = plsc.VectorSubcoreMesh(core_axis_name="c", subcore_axis_name="s",
                                  num_cores=NC, num_subcores=NS)
    fn = pl.kernel(
        _sc_body,
        out_type=[
            jax.ShapeDtypeStruct((NC, NS, NP_ * FB), jnp.float32),
            jax.ShapeDtypeStruct((NP_,), jnp.float32),
            jax.ShapeDtypeStruct((E,), jnp.float32),
        ],
        mesh=mesh,
        scratch_types=[
            pltpu.VMEM((NP_,), jnp.float32),
            pltpu.VMEM((NP_,), jnp.float32),
            pltpu.VMEM((NP_ // NS,), jnp.float32),
            pltpu.VMEM((NP_ * FB,), jnp.float32),
            pltpu.VMEM((NP_ * FB,), jnp.float32),
            pltpu.VMEM((CH,), jnp.int32),
            pltpu.VMEM((CH,), jnp.int32),
            pltpu.VMEM((CH,), jnp.float32),
            pltpu.VMEM((CH,), jnp.int32),
            pltpu.VMEM((CH,), jnp.int32),
            pltpu.VMEM((CH,), jnp.float32),
            pltpu.SemaphoreType.DMA,
            pltpu.SemaphoreType.DMA,
            pltpu.MemorySpace.VMEM_SHARED((NS, NP_), jnp.float32),
            pltpu.MemorySpace.VMEM_SHARED((NP_,), jnp.float32),
        ],
        compiler_params=pltpu.CompilerParams(needs_layout_passes=False),
    )
    return fn(edge_index[0], edge_index[1], edge_attr, xt)


def _tc_body(ax_ref, x_ref, dis_ref, wzc, wzl, bzc, bzl, whc, whl, bhc, bhl,
             att_ref, w1, b1, w2, b2, out_ref, hid_ref):
    parts = ax_ref[...]
    dis = dis_ref[...]
    ax = parts[0] + parts[1] + x_ref[...] * (dis * dis)

    mz = jnp.dot(wzc[...], wzl[...], preferred_element_type=jnp.float32)
    cz = jnp.dot(bzc[...], wzl[...], preferred_element_type=jnp.float32) + bzl[...]
    mh = jnp.dot(whc[...], whl[...], preferred_element_type=jnp.float32)
    ch = jnp.dot(bhc[...], whl[...], preferred_element_type=jnp.float32) + bhl[...]

    a = att_ref[...]
    e = jnp.exp(a - jnp.max(a))
    pr = e / jnp.sum(e)

    hacc = jnp.zeros(hid_ref.shape, jnp.float32)
    for p in range(P):
        axp = ax[:, p * F:(p + 1) * F]
        az = jnp.dot(axp, mz, preferred_element_type=jnp.float32) + cz
        ah = jnp.dot(axp, mh, preferred_element_type=jnp.float32) + ch
        hacc = hacc + pr[0, p] * (1.0 - jax.nn.sigmoid(az)) * jnp.tanh(ah)
    hid_ref[...] = hacc
    h = jax.nn.relu(hacc)
    h = jax.nn.relu(jnp.dot(h, w1[...], preferred_element_type=jnp.float32)
                    + b1[...])
    out_ref[...] = jnp.dot(h, w2[...], preferred_element_type=jnp.float32) \
        + b2[...]


def _tc_dense(axparts, x64, dis, Wz_c, Wz_lt, bz_c, bz_l, Wh_c, Wh_lt,
              bh_c, bh_l, att, W1, b1, W2, b2):
    BN = 1024
    grid = (NP_ // BN,)
    full = lambda shape: pl.BlockSpec(shape, lambda i: (0,) * len(shape))
    return pl.pallas_call(
        _tc_body,
        grid=grid,
        in_specs=[
            pl.BlockSpec((NC, BN, F * P), lambda i: (0, i, 0)),
            pl.BlockSpec((BN, F * P), lambda i: (i, 0)),
            pl.BlockSpec((BN, 1), lambda i: (i, 0)),
            full((F, OUT)), full((OUT, OUT)), full((1, OUT)), full((1, OUT)),
            full((F, OUT)), full((OUT, OUT)), full((1, OUT)), full((1, OUT)),
            full((1, P)),
            full((OUT, HID)), full((1, HID)), full((HID, ODIM)),
            full((1, ODIM)),
        ],
        out_specs=[
            pl.BlockSpec((BN, ODIM), lambda i: (i, 0)),
            pl.BlockSpec((BN, OUT), lambda i: (i, 0)),
        ],
        out_shape=[
            jax.ShapeDtypeStruct((NP_, ODIM), jnp.float32),
            jax.ShapeDtypeStruct((NP_, OUT), jnp.float32),
        ],
    )(axparts, x64, dis, Wz_c, Wz_lt, bz_c, bz_l, Wh_c, Wh_lt, bh_c, bh_l,
      att, W1, b1, W2, b2)


def kernel(x, edge_index, edge_attr, Wz_c, bz_c, Wr_c, br_c, Wh_c, bh_c,
           Wz_l, bz_l, Wr_l, br_l, Wh_l, bh_l, att, W1, b1, W2, b2):
    x64 = jnp.transpose(x, (0, 2, 1)).reshape(N, P * F)
    x64 = jnp.pad(x64, ((0, NP_ - N), (0, 0)))
    xt = x64.reshape(NP_, NS, FB).transpose(1, 2, 0).reshape(NS, NP_ * FB)

    parts, dis, _ = _sc_spmm(edge_index, edge_attr, xt)

    axparts = parts.reshape(NC, NS, FB, NP_).transpose(0, 3, 1, 2) \
        .reshape(NC, NP_, F * P)
    out, hid = _tc_dense(
        axparts, x64, dis.reshape(NP_, 1),
        Wz_c, Wz_l[:OUT], bz_c.reshape(1, OUT), bz_l.reshape(1, OUT),
        Wh_c, Wh_l[:OUT], bh_c.reshape(1, OUT), bh_l.reshape(1, OUT),

        att.reshape(1, P), W1, b1.reshape(1, HID), W2, b2.reshape(1, ODIM))
    return (out[:N], hid[:N])


# speedup vs baseline: 1.5187x; 1.0007x over previous
import functools

import jax
import jax.numpy as jnp
from jax import lax
from jax.experimental import pallas as pl
from jax.experimental.pallas import tpu as pltpu
from jax.experimental.pallas import tpu_sc as plsc

N = 10000
F = 16
P = 4
OUT = 256
HID = 128
ODIM = 12
E = 640000

NP_ = 10240
NROW = NP_ // 16
FB = 4
NC = 2
NS = 16
CH = 2000
MAGIC = 0x5F3759DF


def _fast_rsqrt(d):
    y = plsc.bitcast(MAGIC - (plsc.bitcast(d, jnp.int32) >> 1), jnp.float32)
    for _ in range(3):
        y = y * (1.5 - 0.5 * d * y * y)
    return y


def _sc_body(rowh, colh, ew, xt, out_hbm, dis_hbm, norm_hbm,
             deg_v, dis_v, tmp_v, x_blk, out_blk,
             ra, ca, wa, rb, cb, wb, sema, semb,
             shared_part, shared_sum):
    cid = lax.axis_index("c")
    tid = lax.axis_index("s")
    zero16 = jnp.zeros((16,), jnp.float32)
    nslice = NP_ // NS
    sbase = tid * nslice

    @plsc.parallel_loop(0, NP_ // 16, 1, unroll=8)
    def _(i):
        deg_v[pl.ds(i * 16, 16)] = zero16

    e1base = tid * (E // NS)

    def p1_chunk(k, _):
        pltpu.sync_copy(colh.at[pl.ds(e1base + k * CH, CH)], cb)
        pltpu.sync_copy(ew.at[pl.ds(e1base + k * CH, CH)], wb)

        @plsc.parallel_loop(0, CH // 16, 1, unroll=5)
        def _(g):
            c16 = cb[pl.ds(g * 16, 16)]
            w16 = wb[pl.ds(g * 16, 16)]
            plsc.addupdate_scatter(deg_v, [c16], w16)
        return 0
    lax.fori_loop(0, (E // NS) // CH, p1_chunk, 0)

    pltpu.sync_copy(deg_v, shared_part.at[tid])
    plsc.subcore_barrier()

    def zero_acc(i, _):
        deg_v[pl.ds(sbase + i * 16, 16)] = zero16
        return 0
    lax.fori_loop(0, nslice // 16, zero_acc, 0)
    for k in range(NS):
        pltpu.sync_copy(shared_part.at[k, pl.ds(sbase, nslice)], tmp_v)

        def acc_add(i, _):
            a = deg_v[pl.ds(sbase + i * 16, 16)]
            deg_v[pl.ds(sbase + i * 16, 16)] = a + tmp_v[pl.ds(i * 16, 16)]
            return 0
        lax.fori_loop(0, nslice // 16, acc_add, 0)
    pltpu.sync_copy(deg_v.at[pl.ds(sbase, nslice)],
                    shared_sum.at[pl.ds(sbase, nslice)])
    plsc.subcore_barrier()
    pltpu.sync_copy(shared_sum, deg_v)

    def mk_dis(i, _):
        dis_v[pl.ds(i * 16, 16)] = _fast_rsqrt(deg_v[pl.ds(i * 16, 16)] + 1.0)
        return 0
    lax.fori_loop(0, NP_ // 16, mk_dis, 0)

    @pl.when((tid == 0) & (cid == 0))
    def _():
        pltpu.sync_copy(dis_v, dis_hbm)

    nbase = cid * (E // NC) + tid * (E // (NC * NS))

    def p15_chunk(k, _):
        off = nbase + k * CH
        pltpu.sync_copy(rowh.at[pl.ds(off, CH)], ra)
        pltpu.sync_copy(colh.at[pl.ds(off, CH)], ca)
        pltpu.sync_copy(ew.at[pl.ds(off, CH)], wa)

        @plsc.parallel_loop(0, CH // 16, 1, unroll=5)
        def _(g):
            r16 = ra[pl.ds(g * 16, 16)]
            c16 = ca[pl.ds(g * 16, 16)]
            w16 = wa[pl.ds(g * 16, 16)]
            dr = plsc.load_gather(dis_v, [r16])
            dc = plsc.load_gather(dis_v, [c16])
            wb[pl.ds(g * 16, 16)] = w16 * dr * dc
        pltpu.sync_copy(wb, norm_hbm.at[pl.ds(off, CH)])
        return 0
    lax.fori_loop(0, (E // (NC * NS)) // CH, p15_chunk, 0)
    plsc.subcore_barrier()

    pltpu.sync_copy(xt.at[tid], x_blk)

    @plsc.parallel_loop(0, (NP_ * FB) // 16, 1, unroll=8)
    def _(i):
        out_blk[pl.ds(i * 16, 16)] = zero16

    e2base = cid * (E // NC)
    nch2 = (E // NC) // CH

    def _start(bufs, sem, cidx):
        off = e2base + cidx * CH
        pltpu.async_copy(rowh.at[pl.ds(off, CH)], bufs[0], sem)
        pltpu.async_copy(colh.at[pl.ds(off, CH)], bufs[1], sem)
        pltpu.async_copy(norm_hbm.at[pl.ds(off, CH)], bufs[2], sem)

    def _drain(bufs, sem):
        pltpu.make_async_copy(rowh.at[pl.ds(e2base, CH)], bufs[0], sem).wait()
        pltpu.make_async_copy(colh.at[pl.ds(e2base, CH)], bufs[1], sem).wait()
        pltpu.make_async_copy(ew.at[pl.ds(e2base, CH)], bufs[2], sem).wait()

    def _process(bufs):
        @plsc.parallel_loop(0, CH // 16, 1, unroll=5)
        def _(g):
            r16 = bufs[0][pl.ds(g * 16, 16)]
            c16 = bufs[1][pl.ds(g * 16, 16)]
            nrm = bufs[2][pl.ds(g * 16, 16)]
            for j in range(FB):
                xv = plsc.load_gather(x_blk, [r16 + j * NP_])
                plsc.addupdate_scatter(out_blk, [c16 + j * NP_], xv * nrm)

    bufs_a = (ra, ca, wa)
    bufs_b = (rb, cb, wb)
    _start(bufs_a, sema, 0)

    def p2_pair(k, _):
        c0 = 2 * k
        _start(bufs_b, semb, c0 + 1)
        _drain(bufs_a, sema)
        _process(bufs_a)

        @pl.when(c0 + 2 < nch2)
        def _():
            _start(bufs_a, sema, c0 + 2)
        _drain(bufs_b, semb)
        _process(bufs_b)
        return 0
    lax.fori_loop(0, nch2 // 2, p2_pair, 0)

    pltpu.sync_copy(out_blk, out_hbm.at[cid, tid])


def _sc_spmm(edge_index, edge_attr, xt):
    mesh = plsc.VectorSubcoreMesh(core_axis_name="c", subcore_axis_name="s",
                                  num_cores=NC, num_subcores=NS)
    fn = pl.kernel(
        _sc_body,
        out_type=[
            jax.ShapeDtypeStruct((NC, NS, NP_ * FB), jnp.float32),
            jax.ShapeDtypeStruct((NP_,), jnp.float32),
            jax.ShapeDtypeStruct((E,), jnp.float32),
        ],
        mesh=mesh,
        scratch_types=[
            pltpu.VMEM((NP_,), jnp.float32),
            pltpu.VMEM((NP_,), jnp.float32),
            pltpu.VMEM((NP_ // NS,), jnp.float32),
            pltpu.VMEM((NP_ * FB,), jnp.float32),
            pltpu.VMEM((NP_ * FB,), jnp.float32),
            pltpu.VMEM((CH,), jnp.int32),
            pltpu.VMEM((CH,), jnp.int32),
            pltpu.VMEM((CH,), jnp.float32),
            pltpu.VMEM((CH,), jnp.int32),
            pltpu.VMEM((CH,), jnp.int32),
            pltpu.VMEM((CH,), jnp.float32),
            pltpu.SemaphoreType.DMA,
            pltpu.SemaphoreType.DMA,
            pltpu.MemorySpace.VMEM_SHARED((NS, NP_), jnp.float32),
            pltpu.MemorySpace.VMEM_SHARED((NP_,), jnp.float32),
        ],
        compiler_params=pltpu.CompilerParams(needs_layout_passes=False),
    )
    return fn(edge_index[0], edge_index[1], edge_attr, xt)


def _tc_body(ax_ref, x_ref, dis_ref, wzc, wzl, bzc, bzl, whc, whl, bhc, bhl,
             att_ref, w1, b1, w2, b2, out_ref, hid_ref):
    parts = ax_ref[...]
    dis = dis_ref[...]
    ax = parts[0] + parts[1] + x_ref[...] * (dis * dis)

    mz = jnp.dot(wzc[...], wzl[...], preferred_element_type=jnp.float32)
    cz = jnp.dot(bzc[...], wzl[...], preferred_element_type=jnp.float32) + bzl[...]
    mh = jnp.dot(whc[...], whl[...], preferred_element_type=jnp.float32)
    ch = jnp.dot(bhc[...], whl[...], preferred_element_type=jnp.float32) + bhl[...]

    a = att_ref[...]
    e = jnp.exp(a - jnp.max(a))
    pr = e / jnp.sum(e)

    hacc = jnp.zeros(hid_ref.shape, jnp.float32)
    for p in range(P):
        axp = ax[:, p * F:(p + 1) * F]
        az = jnp.dot(axp, mz, preferred_element_type=jnp.float32) + cz
        ah = jnp.dot(axp, mh, preferred_element_type=jnp.float32) + ch
        hacc = hacc + pr[0, p] * (1.0 - jax.nn.sigmoid(az)) * jnp.tanh(ah)
    hid_ref[...] = hacc
    h = jax.nn.relu(hacc)
    h = jax.nn.relu(jnp.dot(h, w1[...], preferred_element_type=jnp.float32)
                    + b1[...])
    out_ref[...] = jnp.dot(h, w2[...], preferred_element_type=jnp.float32) \
        + b2[...]


def _tc_dense(axparts, x64, dis, Wz_c, Wz_lt, bz_c, bz_l, Wh_c, Wh_lt,
              bh_c, bh_l, att, W1, b1, W2, b2):
    BN = 1024
    grid = (NP_ // BN,)
    full = lambda shape: pl.BlockSpec(shape, lambda i: (0,) * len(shape))
    return pl.pallas_call(
        _tc_body,
        grid=grid,
        in_specs=[
            pl.BlockSpec((NC, BN, F * P), lambda i: (0, i, 0)),
            pl.BlockSpec((BN, F * P), lambda i: (i, 0)),
            pl.BlockSpec((BN, 1), lambda i: (i, 0)),
            full((F, OUT)), full((OUT, OUT)), full((1, OUT)), full((1, OUT)),
            full((F, OUT)), full((OUT, OUT)), full((1, OUT)), full((1, OUT)),
            full((1, P)),
            full((OUT, HID)), full((1, HID)), full((HID, ODIM)),
            full((1, ODIM)),
        ],
        out_specs=[
            pl.BlockSpec((BN, ODIM), lambda i: (i, 0)),
            pl.BlockSpec((BN, OUT), lambda i: (i, 0)),
        ],
        out_shape=[
            jax.ShapeDtypeStruct((NP_, ODIM), jnp.float32),
            jax.ShapeDtypeStruct((NP_, OUT), jnp.float32),
        ],
    )(axparts, x64, dis, Wz_c, Wz_lt, bz_c, bz_l, Wh_c, Wh_lt, bh_c, bh_l,
      att, W1, b1, W2, b2)


def kernel(x, edge_index, edge_attr, Wz_c, bz_c, Wr_c, br_c, Wh_c, bh_c,
           Wz_l, bz_l, Wr_l, br_l, Wh_l, bh_l, att, W1, b1, W2, b2):
    x64 = jnp.transpose(x, (0, 2, 1)).reshape(N, P * F)
    x64 = jnp.pad(x64, ((0, NP_ - N), (0, 0)))
    xt = x64.reshape(NP_, NS, FB).transpose(1, 2, 0).reshape(NS, NP_ * FB)

    parts, dis, _ = _sc_spmm(edge_index, edge_attr, xt)

    axparts = parts.reshape(NC, NS, FB, NP_).transpose(0, 3, 1, 2) \
        .reshape(NC, NP_, F * P)
    out, hid = _tc_dense(
        axparts, x64, dis.reshape(NP_, 1),
        Wz_c, Wz_l[:OUT], bz_c.reshape(1, OUT), bz_l.reshape(1, OUT),
        Wh_c, Wh_l[:OUT], bh_c.reshape(1, OUT), bh_l.reshape(1, OUT),

        att.reshape(1, P), W1, b1.reshape(1, HID), W2, b2.reshape(1, ODIM))
    return (out[:N], hid[:N])
